# baseline (speedup 1.0000x reference)
import functools

import jax
import jax.numpy as jnp
import numpy as np
from jax import lax
from jax.experimental import pallas as pl
from jax.experimental.pallas import tpu as pltpu
from jax.experimental.pallas import tpu_sc as plsc

N = 10000
E = 320000
D = 128

NC = 2
NS = 16
NW = NC * NS

CHUNK = 64
NCHUNK = 168
EPW = NCHUNK * CHUNK
EPAD = NW * EPW
IBLK = 24
NGBUF = 3
NPBUF = 2
ROWS_PT = 624
TAIL = N - NS * ROWS_PT


def _unpack_perm() -> np.ndarray:
    u = np.zeros(D, dtype=np.int64)
    for blk in range(D // 32):
        base = 32 * blk
        for j in range(16):
            u[base + j] = base + 2 * j
            u[base + 16 + j] = base + 2 * j + 1
    return u


_U = _unpack_perm()
_UU = _U[_U]
_Q = np.argsort(_UU)


def _tc_linear_bf16(x, W, b):
    blk = 1000

    def body(x_ref, w_ref, b_ref, o_ref):
        z = (
            lax.dot_general(
                x_ref[...], w_ref[...],
                (((1,), (1,)), ((), ())),
                preferred_element_type=jnp.float32,
            )
            + b_ref[...]
        )
        o_ref[...] = z.astype(jnp.bfloat16)

    return pl.pallas_call(
        body,
        grid=(N // blk,),
        in_specs=[
            pl.BlockSpec((blk, D), lambda i: (i, 0)),
            pl.BlockSpec((D, D), lambda i: (0, 0)),
            pl.BlockSpec((1, D), lambda i: (0, 0)),
        ],
        out_specs=pl.BlockSpec((blk, D), lambda i: (i, 0)),
        out_shape=jax.ShapeDtypeStruct((N, D), jnp.bfloat16),
    )(x, W, b.reshape(1, D))


def _tc_combine(partials, to_bf16):
    blk = 1000
    out_dtype = jnp.bfloat16 if to_bf16 else jnp.float32

    def body(p_ref, o_ref):
        z = p_ref[0] + p_ref[1]
        o_ref[...] = z.astype(out_dtype)

    return pl.pallas_call(
        body,
        grid=(N // blk,),
        in_specs=[pl.BlockSpec((2, blk, D), lambda i: (0, i, 0))],
        out_specs=pl.BlockSpec((blk, D), lambda i: (i, 0)),
        out_shape=jax.ShapeDtypeStruct((N, D), out_dtype),
    )(partials)


def _sc_spmm(z, src3, dst3, w3):
    mesh = plsc.VectorSubcoreMesh(core_axis_name="c", subcore_axis_name="s")

    @functools.partial(
        pl.kernel,
        out_type=jax.ShapeDtypeStruct((NC, N, D), jnp.float32),
        mesh=mesh,
        compiler_params=pltpu.CompilerParams(use_tc_tiling_on_sc=False),
        scratch_types=[
            pltpu.VMEM_SHARED((N, D), jnp.float32),
            pltpu.VMEM((IBLK, CHUNK), jnp.int32),
            pltpu.VMEM((IBLK, CHUNK), jnp.int32),
            pltpu.VMEM((IBLK, CHUNK), jnp.float32),
        ]
        + [pltpu.VMEM((CHUNK, D // 2), jnp.int32)] * NGBUF
        + [pltpu.VMEM((CHUNK, D), jnp.float32)] * NPBUF
        + [pltpu.SemaphoreType.DMA] * (NGBUF + NPBUF),
    )
    def spmm(z_hbm, src_hbm, dst_hbm, w_hbm, out_hbm,
             acc_sh, src_v, dst_v, w_v, *bufs_and_sems):
        gbuf = list(bufs_and_sems[:NGBUF])
        pbuf = list(bufs_and_sems[NGBUF:NGBUF + NPBUF])
        gsem = list(bufs_and_sems[NGBUF + NPBUF:2 * NGBUF + NPBUF])
        ssem = list(bufs_and_sems[2 * NGBUF + NPBUF:])
        cid = lax.axis_index("c")
        sid = lax.axis_index("s")
        wid = cid * NS + sid

        def zrow(r, _):
            for q in range(D // 16):
                pbuf[0][r, pl.ds(q * 16, 16)] = jnp.zeros((16,), jnp.float32)
            return 0
        lax.fori_loop(0, CHUNK, zrow, 0)
        for j in range(ROWS_PT // CHUNK):
            pltpu.sync_copy(pbuf[0],
                            acc_sh.at[pl.ds(sid * ROWS_PT + j * CHUNK, CHUNK)])
        rem = ROWS_PT % CHUNK
        pltpu.sync_copy(
            pbuf[0].at[pl.ds(0, rem)],
            acc_sh.at[pl.ds(sid * ROWS_PT + (ROWS_PT // CHUNK) * CHUNK, rem)])

        @pl.when(sid == NS - 1)
        def _():
            pltpu.sync_copy(pbuf[0].at[pl.ds(0, TAIL)],
                            acc_sh.at[pl.ds(NS * ROWS_PT, TAIL)])
        plsc.subcore_barrier()

        def mult(gb, pb, k):
            def egroup(g, _):
                wv = w_v[k, pl.ds(g * 16, 16)]
                for j in range(16):
                    e = g * 16 + j
                    wt = wv[j]
                    for q in range(D // 32):
                        w32 = gb[e, pl.ds(q * 16, 16)]
                        a = lax.bitcast_convert_type(
                            w32 << 16, jnp.float32)
                        bb = lax.bitcast_convert_type(
                            w32 & jnp.int32(-65536), jnp.float32)
                        pb[e, pl.ds(q * 32, 16)] = a * wt
                        pb[e, pl.ds(q * 32 + 16, 16)] = bb * wt
                return 0
            lax.fori_loop(0, CHUNK // 16, egroup, 0)

        def gather(k, b):
            return pltpu.async_copy(z_hbm.at[src_v.at[k]], gbuf[b], gsem[b])

        def gather_wait(k, b):
            pltpu.make_async_copy(z_hbm.at[src_v.at[k]], gbuf[b],
                                  gsem[b]).wait()

        def scatter(k, b):
            return pltpu.async_copy(pbuf[b], acc_sh.at[dst_v.at[k]], ssem[b],
                                    add=True)

        def scatter_wait(k, b):
            pltpu.make_async_copy(pbuf[b], acc_sh.at[dst_v.at[k]],
                                  ssem[b]).wait()

        def sblock(s, _):
            bsl = pl.ds(s * IBLK, IBLK)
            pltpu.sync_copy(src_hbm.at[wid, bsl], src_v)
            pltpu.sync_copy(dst_hbm.at[wid, bsl], dst_v)
            pltpu.sync_copy(w_hbm.at[wid, bsl], w_v)

            gather(0, 0)
            gather(1, 1)

            def hexa(q, _):
                for j in range(6):
                    kk = q * 6 + j
                    gj = j % NGBUF
                    pj = j % NPBUF
                    gather_wait(kk, gj)

                    @pl.when(kk >= NPBUF)
                    def _():
                        scatter_wait(kk - NPBUF, pj)

                    mult(gbuf[gj], pbuf[pj], kk)
                    scatter(kk, pj)

                    @pl.when(kk <= IBLK - NGBUF)
                    def _():
                        gather(kk + NGBUF - 1, (j + NGBUF - 1) % NGBUF)
                return 0
            lax.fori_loop(0, IBLK // 6, hexa, 0)

            for j in range(NPBUF):
                scatter_wait(IBLK - NPBUF + j, (IBLK - NPBUF + j) % NPBUF)
            return 0
        lax.fori_loop(0, NCHUNK // IBLK, sblock, 0)

        plsc.subcore_barrier()
        sl = pl.ds(sid * ROWS_PT, ROWS_PT)
        pltpu.sync_copy(acc_sh.at[sl], out_hbm.at[cid, sl])

        @pl.when(sid == NS - 1)
        def _():
            tl = pl.ds(NS * ROWS_PT, TAIL)
            pltpu.sync_copy(acc_sh.at[tl], out_hbm.at[cid, tl])

    return spmm(z, src3, dst3, w3)


def kernel(x, edge_index, edge_weight, W, b):
    pad = EPAD - E
    pad_idx = jnp.arange(pad, dtype=jnp.int32) % N
    src3 = jnp.concatenate([edge_index[0], pad_idx]).reshape(NW, NCHUNK, CHUNK)
    dst3 = jnp.concatenate([edge_index[1], pad_idx]).reshape(NW, NCHUNK, CHUNK)
    w3 = jnp.concatenate(
        [edge_weight, jnp.zeros((pad,), jnp.float32)]).reshape(NW, NCHUNK, CHUNK)

    def pack_i32(zbf):
        return lax.bitcast_convert_type(
            zbf.reshape(N, D // 2, 2), jnp.int32)

    q = jnp.asarray(_Q)
    z = pack_i32(_tc_linear_bf16(x, W[q, :], b[q]))
    partials = _sc_spmm(z, src3, dst3, w3)
    z = pack_i32(_tc_combine(partials, to_bf16=True))
    partials = _sc_spmm(z, src3, dst3, w3)
    return _tc_combine(partials, to_bf16=False)


# speedup vs baseline: 2.0144x; 2.0144x over previous
import functools

import jax
import jax.numpy as jnp
from jax import lax
from jax.experimental import pallas as pl
from jax.experimental.pallas import tpu as pltpu
from jax.experimental.pallas import tpu_sc as plsc

N = 10000
E = 320000
D = 128

NC = 2
NS = 16
NW = NC * NS

CHUNK = 80
NCHUNK = 128
EPW = NCHUNK * CHUNK
EPAD = NW * EPW
IBLK = 16
NBUF = 4
ROWS_PT = 624
TAIL = N - NS * ROWS_PT
ZR = 16


def _tc_linear(x, W, b):
    blk = 1000

    def body(x_ref, w_ref, b_ref, o_ref):
        o_ref[...] = (
            lax.dot_general(
                x_ref[...], w_ref[...],
                (((1,), (1,)), ((), ())),
                preferred_element_type=jnp.float32,
            )
            + b_ref[...]
        )

    return pl.pallas_call(
        body,
        grid=(N // blk,),
        in_specs=[
            pl.BlockSpec((blk, D), lambda i: (i, 0)),
            pl.BlockSpec((D, D), lambda i: (0, 0)),
            pl.BlockSpec((1, D), lambda i: (0, 0)),
        ],
        out_specs=pl.BlockSpec((blk, D), lambda i: (i, 0)),
        out_shape=jax.ShapeDtypeStruct((N, D), jnp.float32),
    )(x, W, b.reshape(1, D))


def _tc_combine(partials):
    blk = 1000

    def body(p_ref, o_ref):
        o_ref[...] = p_ref[0] + p_ref[1]

    return pl.pallas_call(
        body,
        grid=(N // blk,),
        in_specs=[pl.BlockSpec((2, blk, D), lambda i: (0, i, 0))],
        out_specs=pl.BlockSpec((blk, D), lambda i: (i, 0)),
        out_shape=jax.ShapeDtypeStruct((N, D), jnp.float32),
    )(partials)


def _sc_spmm(z, src3, dst3, w3):
    mesh = plsc.VectorSubcoreMesh(core_axis_name="c", subcore_axis_name="s")

    @functools.partial(
        pl.kernel,
        out_type=jax.ShapeDtypeStruct((NC, N, D), jnp.float32),
        mesh=mesh,
        scratch_types=[
            pltpu.VMEM_SHARED((N, D), jnp.float32),
            pltpu.VMEM((IBLK, CHUNK), jnp.int32),
            pltpu.VMEM((IBLK, CHUNK), jnp.int32),
            pltpu.VMEM((IBLK, CHUNK), jnp.float32),
        ]
        + [pltpu.VMEM((CHUNK, D), jnp.float32)] * NBUF
        + [pltpu.SemaphoreType.DMA] * (2 * NBUF),
    )
    def spmm(z_hbm, src_hbm, dst_hbm, w_hbm, out_hbm,
             acc_sh, src_v, dst_v, w_v, *bufs_and_sems):
        rows = list(bufs_and_sems[:NBUF])
        gsem = list(bufs_and_sems[NBUF:2 * NBUF])
        ssem = list(bufs_and_sems[2 * NBUF:])
        cid = lax.axis_index("c")
        sid = lax.axis_index("s")
        wid = cid * NS + sid

        def zrow(r, _):
            for q in range(D // 16):
                rows[0][r, pl.ds(q * 16, 16)] = jnp.zeros((16,), jnp.float32)
            return 0
        lax.fori_loop(0, CHUNK, zrow, 0)
        for j in range(ROWS_PT // CHUNK):
            pltpu.sync_copy(rows[0],
                            acc_sh.at[pl.ds(sid * ROWS_PT + j * CHUNK, CHUNK)])
        rem = ROWS_PT % CHUNK
        pltpu.sync_copy(
            rows[0].at[pl.ds(0, rem)],
            acc_sh.at[pl.ds(sid * ROWS_PT + (ROWS_PT // CHUNK) * CHUNK, rem)])

        @pl.when(sid == NS - 1)
        def _():
            pltpu.sync_copy(rows[0].at[pl.ds(0, TAIL)],
                            acc_sh.at[pl.ds(NS * ROWS_PT, TAIL)])
        plsc.subcore_barrier()

        def mult(rv, k):
            def egroup(g, _):
                wv = w_v[k, pl.ds(g * 16, 16)]
                for j in range(16):
                    e = g * 16 + j
                    wt = wv[j]
                    for q in range(D // 16):
                        sl = pl.ds(q * 16, 16)
                        rv[e, sl] = rv[e, sl] * wt
                return 0
            lax.fori_loop(0, CHUNK // 16, egroup, 0)

        H = CHUNK // 2

        def gather(k, b):
            pltpu.async_copy(z_hbm.at[src_v.at[k, pl.ds(0, H)]],
                             rows[b].at[pl.ds(0, H)], gsem[b])
            pltpu.async_copy(z_hbm.at[src_v.at[k, pl.ds(H, H)]],
                             rows[b].at[pl.ds(H, H)], gsem[b])

        def gather_wait(k, b):
            pltpu.make_async_copy(z_hbm.at[src_v.at[k, pl.ds(0, H)]],
                                  rows[b].at[pl.ds(0, H)], gsem[b]).wait()
            pltpu.make_async_copy(z_hbm.at[src_v.at[k, pl.ds(H, H)]],
                                  rows[b].at[pl.ds(H, H)], gsem[b]).wait()

        def scatter(k, b):
            return pltpu.async_copy(rows[b], acc_sh.at[dst_v.at[k]], ssem[b],
                                    add=True)

        def scatter_wait(k, b):
            pltpu.make_async_copy(rows[b], acc_sh.at[dst_v.at[k]],
                                  ssem[b]).wait()

        def sblock(s, _):
            bsl = pl.ds(s * IBLK, IBLK)
            pltpu.sync_copy(src_hbm.at[wid, bsl], src_v)
            pltpu.sync_copy(dst_hbm.at[wid, bsl], dst_v)
            pltpu.sync_copy(w_hbm.at[wid, bsl], w_v)

            gather(0, 0)
            gather(1, 1)

            def quad(q, _):
                for j in range(NBUF):
                    kk = q * NBUF + j
                    j2 = (j + 2) % NBUF
                    gather_wait(kk, j)
                    mult(rows[j], kk)
                    scatter(kk, j)

                    @pl.when(jnp.logical_and(kk >= 2, kk <= IBLK - 3))
                    def _():
                        scatter_wait(kk - 2, j2)
                        gather(kk + 2, j2)

                    @pl.when(kk < 2)
                    def _():
                        gather(kk + 2, j2)
                return 0
            lax.fori_loop(0, IBLK // NBUF, quad, 0)

            for j in range(NBUF):
                scatter_wait(IBLK - NBUF + j, (IBLK - NBUF + j) % NBUF)
            return 0
        lax.fori_loop(0, NCHUNK // IBLK, sblock, 0)

        plsc.subcore_barrier()
        sl = pl.ds(sid * ROWS_PT, ROWS_PT)
        pltpu.sync_copy(acc_sh.at[sl], out_hbm.at[cid, sl])

        @pl.when(sid == NS - 1)
        def _():
            tl = pl.ds(NS * ROWS_PT, TAIL)
            pltpu.sync_copy(acc_sh.at[tl], out_hbm.at[cid, tl])

    return spmm(z, src3, dst3, w3)


def kernel(x, edge_index, edge_weight, W, b):
    pad = EPAD - E
    pad_idx = jnp.arange(pad, dtype=jnp.int32) % N
    src3 = jnp.concatenate([edge_index[0], pad_idx]).reshape(NW, NCHUNK, CHUNK)
    dst3 = jnp.concatenate([edge_index[1], pad_idx]).reshape(NW, NCHUNK, CHUNK)
    w3 = jnp.concatenate(
        [edge_weight, jnp.zeros((pad,), jnp.float32)]).reshape(NW, NCHUNK, CHUNK)

    z = _tc_linear(x, W, b)
    for _ in range(2):
        partials = _sc_spmm(z, src3, dst3, w3)
        z = _tc_combine(partials)
    return z


# speedup vs baseline: 2.2468x; 1.1153x over previous
import functools

import jax
import jax.numpy as jnp
from jax import lax
from jax.experimental import pallas as pl
from jax.experimental.pallas import tpu as pltpu
from jax.experimental.pallas import tpu_sc as plsc

N = 10000
E = 320000
D = 128

NC = 2
NS = 16
NW = NC * NS

CHUNK = 80
NCHUNK = 128
EPW = NCHUNK * CHUNK
EPAD = NW * EPW
IBLK = 8
NBLK = 16
NBUF = 4
ROWS_PT = 624
TAIL = N - NS * ROWS_PT
ZR = 16


def _tc_linear(x, W, b):
    blk = 1000

    def body(x_ref, w_ref, b_ref, o_ref):
        o_ref[...] = (
            lax.dot_general(
                x_ref[...], w_ref[...],
                (((1,), (1,)), ((), ())),
                preferred_element_type=jnp.float32,
            )
            + b_ref[...]
        )

    return pl.pallas_call(
        body,
        grid=(N // blk,),
        in_specs=[
            pl.BlockSpec((blk, D), lambda i: (i, 0)),
            pl.BlockSpec((D, D), lambda i: (0, 0)),
            pl.BlockSpec((1, D), lambda i: (0, 0)),
        ],
        out_specs=pl.BlockSpec((blk, D), lambda i: (i, 0)),
        out_shape=jax.ShapeDtypeStruct((N, D), jnp.float32),
    )(x, W, b.reshape(1, D))


def _tc_combine(partials):
    blk = 1000

    def body(p_ref, o_ref):
        o_ref[...] = p_ref[0] + p_ref[1]

    return pl.pallas_call(
        body,
        grid=(N // blk,),
        in_specs=[pl.BlockSpec((2, blk, D), lambda i: (0, i, 0))],
        out_specs=pl.BlockSpec((blk, D), lambda i: (i, 0)),
        out_shape=jax.ShapeDtypeStruct((N, D), jnp.float32),
    )(partials)


def _sc_spmm(z, src3, dst3, w3):
    mesh = plsc.VectorSubcoreMesh(core_axis_name="c", subcore_axis_name="s")

    @functools.partial(
        pl.kernel,
        out_type=jax.ShapeDtypeStruct((NC, N, D), jnp.float32),
        mesh=mesh,
        scratch_types=[
            pltpu.VMEM_SHARED((N, D), jnp.float32),
            pltpu.VMEM((2, IBLK, CHUNK), jnp.int32),
            pltpu.VMEM((2, IBLK, CHUNK), jnp.int32),
            pltpu.VMEM((2, IBLK, CHUNK), jnp.float32),
        ]
        + [pltpu.VMEM((CHUNK, D), jnp.float32)] * NBUF
        + [pltpu.SemaphoreType.DMA] * (2 * NBUF + 1),
    )
    def spmm(z_hbm, src_hbm, dst_hbm, w_hbm, out_hbm,
             acc_sh, src_v, dst_v, w_v, *bufs_and_sems):
        rows = list(bufs_and_sems[:NBUF])
        gsem = list(bufs_and_sems[NBUF:2 * NBUF])
        ssem = list(bufs_and_sems[2 * NBUF:3 * NBUF])
        stsem = bufs_and_sems[3 * NBUF]
        cid = lax.axis_index("c")
        sid = lax.axis_index("s")
        wid = cid * NS + sid

        def zrow(r, _):
            for q in range(D // 16):
                rows[0][r, pl.ds(q * 16, 16)] = jnp.zeros((16,), jnp.float32)
            return 0
        lax.fori_loop(0, CHUNK, zrow, 0)
        for j in range(ROWS_PT // CHUNK):
            pltpu.sync_copy(rows[0],
                            acc_sh.at[pl.ds(sid * ROWS_PT + j * CHUNK, CHUNK)])
        rem = ROWS_PT % CHUNK
        pltpu.sync_copy(
            rows[0].at[pl.ds(0, rem)],
            acc_sh.at[pl.ds(sid * ROWS_PT + (ROWS_PT // CHUNK) * CHUNK, rem)])

        @pl.when(sid == NS - 1)
        def _():
            pltpu.sync_copy(rows[0].at[pl.ds(0, TAIL)],
                            acc_sh.at[pl.ds(NS * ROWS_PT, TAIL)])
        plsc.subcore_barrier()

        def mult(rv, k):
            tp = (k // IBLK) % 2
            tr = k % IBLK

            def egroup(g, _):
                wv = w_v[tp, tr, pl.ds(g * 16, 16)]
                for j in range(16):
                    e = g * 16 + j
                    wt = wv[j]
                    for q in range(D // 16):
                        sl = pl.ds(q * 16, 16)
                        rv[e, sl] = rv[e, sl] * wt
                return 0
            lax.fori_loop(0, CHUNK // 16, egroup, 0)

        def pr(k):
            return (k // IBLK) % 2, k % IBLK

        def gather(k, b):
            p, r = pr(k)
            pltpu.async_copy(z_hbm.at[src_v.at[p, r]], rows[b], gsem[b])

        def gather_wait(k, b):
            p, r = pr(k)
            pltpu.make_async_copy(z_hbm.at[src_v.at[p, r]], rows[b],
                                  gsem[b]).wait()

        def scatter(k, b):
            p, r = pr(k)
            pltpu.async_copy(rows[b], acc_sh.at[dst_v.at[p, r]], ssem[b],
                             add=True)

        def scatter_wait(k, b):
            p, r = pr(k)
            pltpu.make_async_copy(rows[b], acc_sh.at[dst_v.at[p, r]],
                                  ssem[b]).wait()

        def stage_issue(t):
            pp = t % 2
            bsl = pl.ds(t * IBLK, IBLK)
            pltpu.async_copy(src_hbm.at[wid, bsl], src_v.at[pp], stsem)
            pltpu.async_copy(dst_hbm.at[wid, bsl], dst_v.at[pp], stsem)
            pltpu.async_copy(w_hbm.at[wid, bsl], w_v.at[pp], stsem)

        def stage_wait(t):
            pp = t % 2
            bsl = pl.ds(t * IBLK, IBLK)
            pltpu.make_async_copy(src_hbm.at[wid, bsl], src_v.at[pp],
                                  stsem).wait()
            pltpu.make_async_copy(dst_hbm.at[wid, bsl], dst_v.at[pp],
                                  stsem).wait()
            pltpu.make_async_copy(w_hbm.at[wid, bsl], w_v.at[pp],
                                  stsem).wait()

        bsl0 = pl.ds(0, IBLK)
        pltpu.sync_copy(src_hbm.at[wid, bsl0], src_v.at[0])
        pltpu.sync_copy(dst_hbm.at[wid, bsl0], dst_v.at[0])
        pltpu.sync_copy(w_hbm.at[wid, bsl0], w_v.at[0])
        gather(0, 0)
        gather(1, 1)

        def quad(q, _):
            for j in range(NBUF):
                kk = q * NBUF + j
                j2 = (j + 2) % NBUF
                t = kk // IBLK
                gather_wait(kk, j)

                if j == 0:
                    @pl.when(jnp.logical_and(kk % IBLK == 4, t <= NBLK - 2))
                    def _():
                        stage_wait(t + 1)

                mult(rows[j], kk)
                scatter(kk, j)

                if j == 2:
                    @pl.when(jnp.logical_and(kk % IBLK == 2, t <= NBLK - 2))
                    def _():
                        stage_issue(t + 1)

                @pl.when(jnp.logical_and(kk >= 2, kk <= NCHUNK - 3))
                def _():
                    scatter_wait(kk - 2, j2)
                    gather(kk + 2, j2)

                @pl.when(kk < 2)
                def _():
                    gather(kk + 2, j2)
            return 0
        lax.fori_loop(0, NCHUNK // NBUF, quad, 0)

        for j in range(NBUF):
            scatter_wait(NCHUNK - NBUF + j, (NCHUNK - NBUF + j) % NBUF)

        plsc.subcore_barrier()
        sl = pl.ds(sid * ROWS_PT, ROWS_PT)
        pltpu.sync_copy(acc_sh.at[sl], out_hbm.at[cid, sl])

        @pl.when(sid == NS - 1)
        def _():
            tl = pl.ds(NS * ROWS_PT, TAIL)
            pltpu.sync_copy(acc_sh.at[tl], out_hbm.at[cid, tl])

    return spmm(z, src3, dst3, w3)


def kernel(x, edge_index, edge_weight, W, b):
    pad = EPAD - E
    pad_idx = jnp.arange(pad, dtype=jnp.int32) % N
    src3 = jnp.concatenate([edge_index[0], pad_idx]).reshape(NW, NCHUNK, CHUNK)
    dst3 = jnp.concatenate([edge_index[1], pad_idx]).reshape(NW, NCHUNK, CHUNK)
    w3 = jnp.concatenate(
        [edge_weight, jnp.zeros((pad,), jnp.float32)]).reshape(NW, NCHUNK, CHUNK)

    z = _tc_linear(x, W, b)
    for _ in range(2):
        partials = _sc_spmm(z, src3, dst3, w3)
        z = _tc_combine(partials)
    return z
